# baseline (speedup 1.0000x reference)
import jax
import jax.numpy as jnp
from jax.experimental import pallas as pl


def _copy_body(in_ref, out_ref):
    out_ref[...] = in_ref[...]


def _copy(x, block_rows):
    n_rows, width = x.shape
    grid = n_rows // block_rows
    return pl.pallas_call(
        _copy_body,
        out_shape=jax.ShapeDtypeStruct(x.shape, x.dtype),
        grid=(grid,),
        in_specs=[pl.BlockSpec((block_rows, width), lambda i: (i, 0))],
        out_specs=pl.BlockSpec((block_rows, width), lambda i: (i, 0)),
    )(x)


def kernel(emb_user, emb_item):
    out_user = _copy(emb_user, 8000)
    out_item = _copy(emb_item, 10000)
    return (out_user, out_item)
